# baseline (speedup 1.0000x reference)
import functools

import numpy as np

import jax
import jax.numpy as jnp
from jax import lax
from jax.experimental import pallas as pl
from jax.experimental.pallas import tpu as pltpu
from jax.experimental.pallas import tpu_sc as plsc

_NW = 32
_CHUNK = 32
_LW = 128


def _sc_gather(XiW, tables, B, C):
    n_t = len(tables)
    mesh = plsc.VectorSubcoreMesh(core_axis_name="c", subcore_axis_name="s")
    scratch = [
        pltpu.VMEM((n_t, _CHUNK), jnp.int32),
        pltpu.VMEM((_CHUNK, n_t * _LW), jnp.float32),
        pltpu.SemaphoreType.DMA,
        pltpu.SemaphoreType.DMA,
        pltpu.SemaphoreType.DMA,
    ]

    def body(xiw_hbm, *refs):
        tabs = refs[:n_t]
        out_hbm = refs[n_t]
        idx_v = refs[n_t + 1]
        packed = refs[n_t + 2]
        sem_i, sem_g, sem_o = refs[n_t + 3:]
        w = lax.axis_index("s") * 2 + lax.axis_index("c")

        @pl.loop(0, C)
        def _(c):
            base = w * (C * _CHUNK) + c * _CHUNK
            pltpu.async_copy(xiw_hbm.at[w, c], idx_v, sem_i).wait()
            gs = [pltpu.async_copy(
                tabs[i].at[idx_v.at[i]],
                packed.at[:, pl.ds(i * _LW, _LW)],
                sem_g) for i in range(n_t)]
            for g in gs:
                g.wait()
            pltpu.async_copy(packed, out_hbm.at[pl.ds(base, _CHUNK)],
                             sem_o).wait()

    kern = pl.kernel(
        body,
        out_type=jax.ShapeDtypeStruct((B, n_t * _LW), jnp.float32),
        mesh=mesh,
        scratch_types=scratch,
    )
    return kern(XiW, *tables)


def _tc_dense(G, Xi, Xv, svec, pts, bws, bbs, a1, tb1, w2, tb2, w3, tb3,
              w4, tb4, B, NF):
    BT = 512
    bf16 = jnp.bfloat16
    f32 = jnp.float32
    n1 = NF + 1
    GW = G.shape[1]

    def body(g_ref, xi_ref, xv_ref, pt_ref, bw1_ref, bw2_ref, bw3_ref,
             bb1_ref, bb2_ref, bb3_ref, a1_ref, tb1_ref, w2_ref, tb2_ref,
             w3_ref, tb3_ref, w4_ref, tb4_ref, out_ref):
        h = xv_ref[...].astype(bf16)
        h = jnp.maximum(jnp.dot(h, bw1_ref[...], preferred_element_type=f32)
                        + bb1_ref[...], 0.0)
        h = jnp.maximum(jnp.dot(h.astype(bf16), bw2_ref[...],
                                preferred_element_type=f32) + bb2_ref[...], 0.0)
        h = jnp.maximum(jnp.dot(h.astype(bf16), bw3_ref[...],
                                preferred_element_type=f32) + bb3_ref[...], 0.0)
        g = g_ref[...]
        xi = xi_ref[...]
        pt = pt_ref[...]
        lane = lax.broadcasted_iota(jnp.int32, (1, _LW), 1)
        embs = []
        for i in range(NF):
            s = int(svec[i])
            seg = lane // (_LW // s)
            p = lax.rem(xi[:, i:i + 1], s)
            stripe = g[:, i * _LW:(i + 1) * _LW]
            gsel = jnp.where(seg == p, stripe, 0.0).astype(bf16)
            embs.append(jnp.dot(gsel, pt[i * _LW:(i + 1) * _LW, :],
                                preferred_element_type=f32))
        t = jnp.concatenate([h] + embs, axis=1).astype(bf16)
        t3 = t.reshape(BT, n1, 64)
        z3 = lax.dot_general(t3, t3, (((2,), (2,)), ((0,), (0,))),
                             preferred_element_type=f32)
        zr = z3.reshape(BT, n1 * n1).astype(bf16)
        x1 = jnp.concatenate([h.astype(bf16), zr], axis=1)
        y = jnp.maximum(jnp.dot(x1, a1_ref[...], preferred_element_type=f32)
                        + tb1_ref[...], 0.0)
        y = jnp.maximum(jnp.dot(y.astype(bf16), w2_ref[...],
                                preferred_element_type=f32) + tb2_ref[...], 0.0)
        y = jnp.maximum(jnp.dot(y.astype(bf16), w3_ref[...],
                                preferred_element_type=f32) + tb3_ref[...], 0.0)
        y = jnp.dot(y.astype(bf16), w4_ref[...], preferred_element_type=f32) \
            + tb4_ref[...]
        out_ref[...] = jax.nn.sigmoid(y)

    grid = (B // BT,)
    full = lambda a: pl.BlockSpec(a.shape, lambda i: (0,) * a.ndim)
    in_specs = [
        pl.BlockSpec((BT, GW), lambda i: (i, 0)),
        pl.BlockSpec((BT, NF), lambda i: (i, 0)),
        pl.BlockSpec((BT, Xv.shape[1]), lambda i: (i, 0)),
        full(pts),
        full(bws[0]), full(bws[1]), full(bws[2]),
        full(bbs[0]), full(bbs[1]), full(bbs[2]),
        full(a1), full(tb1), full(w2), full(tb2), full(w3), full(tb3),
        full(w4), full(tb4),
    ]
    out_spec = pl.BlockSpec((BT, 1), lambda i: (i, 0))
    return pl.pallas_call(
        body,
        grid=grid,
        in_specs=in_specs,
        out_specs=out_spec,
        out_shape=jax.ShapeDtypeStruct((B, 1), jnp.float32),
    )(G, Xi, Xv, pts, *bws, *bbs, a1, tb1, w2, tb2, w3, tb3, w4, tb4)


def kernel(Xi, Xv, tables, projs, bot_w, bot_b, top_w, top_b):
    B, NF = Xi.shape
    md = [int(t.shape[1]) for t in tables]
    EMB = projs[0].shape[0]
    n1 = NF + 1
    C = B // (_NW * _CHUNK)

    svec = [8 if m <= _LW // 8 else (4 if m <= _LW // 4 else 2) for m in md]

    sarr = jnp.asarray(svec, dtype=jnp.int32)[None, :]
    XiS = Xi // sarr
    XiW = XiS.T.reshape(NF, _NW, C, _CHUNK).transpose(1, 2, 0, 3)

    def _repack(t, s):
        V, m = t.shape
        W = _LW // s
        return jnp.concatenate(
            [jnp.pad(t[u::s, :], ((0, 0), (0, W - m))) for u in range(s)],
            axis=1)

    tpk = [_repack(t, s) for t, s in zip(tables, svec)]

    bf16 = jnp.bfloat16
    pts = jnp.concatenate(
        [jnp.tile(jnp.pad(p.T.astype(bf16),
                          ((0, _LW // s - p.shape[1]), (0, 0))), (s, 1))
         for p, s in zip(projs, svec)], axis=0)

    bws = [w.T.astype(bf16) for w in bot_w]
    bbs = [b.reshape(1, -1) for b in bot_b]

    W1 = top_w[0]
    li, lj = np.tril_indices(n1, -1)
    rowidx = jnp.asarray(li * n1 + lj, dtype=jnp.int32)
    a_gram = jnp.zeros((n1 * n1, W1.shape[0]), dtype=jnp.float32)
    a_gram = a_gram.at[rowidx].set(W1[:, EMB:].T)
    a1 = jnp.concatenate([W1[:, :EMB].T, a_gram], axis=0).astype(bf16)
    tb1 = top_b[0].reshape(1, -1)
    w2 = top_w[1].T.astype(bf16)
    tb2 = top_b[1].reshape(1, -1)
    w3 = top_w[2].T.astype(bf16)
    tb3 = top_b[2].reshape(1, -1)
    w4 = top_w[3].T.astype(bf16)
    tb4 = top_b[3].reshape(1, -1)

    G = _sc_gather(XiW, tpk, B, C)
    return _tc_dense(G, Xi, Xv, svec, pts, bws, bbs, a1, tb1, w2, tb2, w3,
                     tb3, w4, tb4, B, NF)


# speedup vs baseline: 13.2840x; 13.2840x over previous
import functools

import numpy as np

import jax
import jax.numpy as jnp
from jax import lax
from jax.experimental import pallas as pl
from jax.experimental.pallas import tpu as pltpu
from jax.experimental.pallas import tpu_sc as plsc

_NW = 32
_CHUNK = 32
_LW = 128
_VB = 1024


def _tc_project(tTs, pTs, V, EMB):
    n_t = len(tTs)
    n_p = n_t // 2
    bf16 = jnp.bfloat16
    f32 = jnp.float32

    def body(*refs):
        t_refs = refs[:n_t]
        p_refs = refs[n_t:2 * n_t]
        o_refs = refs[2 * n_t:]
        for p in range(n_p):
            halves = []
            for u in (2 * p, 2 * p + 1):
                tp = lax.dot_general(
                    t_refs[u][...].astype(bf16), p_refs[u][...].astype(bf16),
                    (((0,), (0,)), ((), ())), preferred_element_type=f32)
                halves.append(tp)
            o_refs[p][...] = jnp.concatenate(halves, axis=1)

    grid = (pl.cdiv(V, _VB),)
    in_specs = (
        [pl.BlockSpec((t.shape[0], _VB), lambda i: (0, i)) for t in tTs]
        + [pl.BlockSpec(p.shape, lambda i: (0, 0)) for p in pTs]
    )
    out_specs = [pl.BlockSpec((_VB, _LW), lambda i: (i, 0))
                 for _ in range(n_p)]
    return pl.pallas_call(
        body,
        grid=grid,
        in_specs=in_specs,
        out_specs=out_specs,
        out_shape=[jax.ShapeDtypeStruct((V, _LW), jnp.float32)
                   for _ in range(n_p)],
    )(*tTs, *pTs)


def _sc_gather(XiW, pairs, B, C):
    n_p = len(pairs)
    n_t = 2 * n_p
    mesh = plsc.VectorSubcoreMesh(core_axis_name="c", subcore_axis_name="s")
    scratch = [
        pltpu.VMEM((n_t, _CHUNK), jnp.int32),
        pltpu.VMEM((_CHUNK, n_t * _LW), jnp.float32),
        pltpu.SemaphoreType.DMA,
        pltpu.SemaphoreType.DMA,
        pltpu.SemaphoreType.DMA,
    ]

    def body(xiw_hbm, *refs):
        tabs = refs[:n_p]
        out_hbm = refs[n_p]
        idx_v = refs[n_p + 1]
        packed = refs[n_p + 2]
        sem_i, sem_g, sem_o = refs[n_p + 3:]
        w = lax.axis_index("s") * 2 + lax.axis_index("c")

        @pl.loop(0, C)
        def _(c):
            base = w * (C * _CHUNK) + c * _CHUNK
            pltpu.async_copy(xiw_hbm.at[w, c], idx_v, sem_i).wait()
            gs = [pltpu.async_copy(
                tabs[i // 2].at[idx_v.at[i]],
                packed.at[:, pl.ds(i * _LW, _LW)],
                sem_g) for i in range(n_t)]
            for g in gs:
                g.wait()
            pltpu.async_copy(packed, out_hbm.at[pl.ds(base, _CHUNK)],
                             sem_o).wait()

    kern = pl.kernel(
        body,
        out_type=jax.ShapeDtypeStruct((B, n_t * _LW), jnp.float32),
        mesh=mesh,
        scratch_types=scratch,
    )
    return kern(XiW, *pairs)


def _tc_dense(G, Xv, bws, bbs, a1, tb1, w2, tb2, w3, tb3, w4, tb4, B, NF,
              EMB):
    BT = 512
    bf16 = jnp.bfloat16
    f32 = jnp.float32
    n1 = NF + 1
    GW = G.shape[1]

    def body(g_ref, xv_ref, bw1_ref, bw2_ref, bw3_ref,
             bb1_ref, bb2_ref, bb3_ref, a1_ref, tb1_ref, w2_ref, tb2_ref,
             w3_ref, tb3_ref, w4_ref, tb4_ref, out_ref):
        h = xv_ref[...].astype(bf16)
        h = jnp.maximum(jnp.dot(h, bw1_ref[...], preferred_element_type=f32)
                        + bb1_ref[...], 0.0)
        h = jnp.maximum(jnp.dot(h.astype(bf16), bw2_ref[...],
                                preferred_element_type=f32) + bb2_ref[...], 0.0)
        h = jnp.maximum(jnp.dot(h.astype(bf16), bw3_ref[...],
                                preferred_element_type=f32) + bb3_ref[...], 0.0)
        g = g_ref[...]
        embs = [g[:, i * _LW + (i % 2) * EMB: i * _LW + (i % 2) * EMB + EMB]
                for i in range(NF)]
        t = jnp.concatenate([h] + embs, axis=1).astype(bf16)
        t3 = t.reshape(BT, n1, EMB)
        z3 = lax.dot_general(t3, t3, (((2,), (2,)), ((0,), (0,))),
                             preferred_element_type=f32)
        zr = z3.reshape(BT, n1 * n1).astype(bf16)
        x1 = jnp.concatenate([h.astype(bf16), zr], axis=1)
        y = jnp.maximum(jnp.dot(x1, a1_ref[...], preferred_element_type=f32)
                        + tb1_ref[...], 0.0)
        y = jnp.maximum(jnp.dot(y.astype(bf16), w2_ref[...],
                                preferred_element_type=f32) + tb2_ref[...], 0.0)
        y = jnp.maximum(jnp.dot(y.astype(bf16), w3_ref[...],
                                preferred_element_type=f32) + tb3_ref[...], 0.0)
        y = jnp.dot(y.astype(bf16), w4_ref[...], preferred_element_type=f32) \
            + tb4_ref[...]
        out_ref[...] = jax.nn.sigmoid(y)

    grid = (B // BT,)
    full = lambda a: pl.BlockSpec(a.shape, lambda i: (0,) * a.ndim)
    in_specs = [
        pl.BlockSpec((BT, GW), lambda i: (i, 0)),
        pl.BlockSpec((BT, Xv.shape[1]), lambda i: (i, 0)),
        full(bws[0]), full(bws[1]), full(bws[2]),
        full(bbs[0]), full(bbs[1]), full(bbs[2]),
        full(a1), full(tb1), full(w2), full(tb2), full(w3), full(tb3),
        full(w4), full(tb4),
    ]
    out_spec = pl.BlockSpec((BT, 1), lambda i: (i, 0))
    return pl.pallas_call(
        body,
        grid=grid,
        in_specs=in_specs,
        out_specs=out_spec,
        out_shape=jax.ShapeDtypeStruct((B, 1), jnp.float32),
    )(G, Xv, *bws, *bbs, a1, tb1, w2, tb2, w3, tb3, w4, tb4)


def kernel(Xi, Xv, tables, projs, bot_w, bot_b, top_w, top_b):
    B, NF = Xi.shape
    V = tables[0].shape[0]
    EMB = projs[0].shape[0]
    n1 = NF + 1
    C = B // (_NW * _CHUNK)

    XiW = Xi.T.reshape(NF, _NW, C, _CHUNK).transpose(1, 2, 0, 3)
    tTs = [t.T for t in tables]
    pTs = [p.T for p in projs]

    bf16 = jnp.bfloat16
    bws = [w.T.astype(bf16) for w in bot_w]
    bbs = [b.reshape(1, -1) for b in bot_b]

    W1 = top_w[0]
    li, lj = np.tril_indices(n1, -1)
    rowidx = jnp.asarray(li * n1 + lj, dtype=jnp.int32)
    a_gram = jnp.zeros((n1 * n1, W1.shape[0]), dtype=jnp.float32)
    a_gram = a_gram.at[rowidx].set(W1[:, EMB:].T)
    a1 = jnp.concatenate([W1[:, :EMB].T, a_gram], axis=0).astype(bf16)
    tb1 = top_b[0].reshape(1, -1)
    w2 = top_w[1].T.astype(bf16)
    tb2 = top_b[1].reshape(1, -1)
    w3 = top_w[2].T.astype(bf16)
    tb3 = top_b[2].reshape(1, -1)
    w4 = top_w[3].T.astype(bf16)
    tb4 = top_b[3].reshape(1, -1)

    pairs = _tc_project(tTs, pTs, V, EMB)
    G = _sc_gather(XiW, pairs, B, C)
    return _tc_dense(G, Xv, bws, bbs, a1, tb1, w2, tb2, w3, tb3, w4, tb4,
                     B, NF, EMB)


# speedup vs baseline: 13.6513x; 1.0277x over previous
import functools

import numpy as np

import jax
import jax.numpy as jnp
from jax import lax
from jax.experimental import pallas as pl
from jax.experimental.pallas import tpu as pltpu
from jax.experimental.pallas import tpu_sc as plsc

_NW = 32
_CHUNK = 32
_LW = 128
_VB = 2048


def _tc_project(tTs, pTs, V, EMB):
    n_t = len(tTs)
    n_p = n_t // 2
    bf16 = jnp.bfloat16
    f32 = jnp.float32

    def body(*refs):
        t_refs = refs[:n_t]
        p_refs = refs[n_t:2 * n_t]
        o_refs = refs[2 * n_t:]
        for p in range(n_p):
            halves = []
            for u in (2 * p, 2 * p + 1):
                tp = lax.dot_general(
                    t_refs[u][...].astype(bf16), p_refs[u][...].astype(bf16),
                    (((0,), (0,)), ((), ())), preferred_element_type=f32)
                halves.append(tp)
            o_refs[p][...] = jnp.concatenate(halves, axis=1)

    grid = (pl.cdiv(V, _VB),)
    in_specs = (
        [pl.BlockSpec((t.shape[0], _VB), lambda i: (0, i)) for t in tTs]
        + [pl.BlockSpec(p.shape, lambda i: (0, 0)) for p in pTs]
    )
    out_specs = [pl.BlockSpec((_VB, _LW), lambda i: (i, 0))
                 for _ in range(n_p)]
    return pl.pallas_call(
        body,
        grid=grid,
        in_specs=in_specs,
        out_specs=out_specs,
        out_shape=[jax.ShapeDtypeStruct((V, _LW), jnp.float32)
                   for _ in range(n_p)],
        compiler_params=pltpu.CompilerParams(
            vmem_limit_bytes=100 * 1024 * 1024),
    )(*tTs, *pTs)


def _sc_gather(XiW, pairs, B, C):
    n_p = len(pairs)
    n_t = 2 * n_p
    mesh = plsc.VectorSubcoreMesh(core_axis_name="c", subcore_axis_name="s")
    scratch = [
        pltpu.VMEM((n_t, _CHUNK), jnp.int32),
        pltpu.VMEM((_CHUNK, n_t * _LW), jnp.float32),
        pltpu.SemaphoreType.DMA,
        pltpu.SemaphoreType.DMA,
        pltpu.SemaphoreType.DMA,
    ]

    def body(xiw_hbm, *refs):
        tabs = refs[:n_p]
        out_hbm = refs[n_p]
        idx_v = refs[n_p + 1]
        packed = refs[n_p + 2]
        sem_i, sem_g, sem_o = refs[n_p + 3:]
        w = lax.axis_index("s") * 2 + lax.axis_index("c")

        @pl.loop(0, C)
        def _(c):
            base = w * (C * _CHUNK) + c * _CHUNK
            pltpu.async_copy(xiw_hbm.at[w, c], idx_v, sem_i).wait()
            gs = [pltpu.async_copy(
                tabs[i // 2].at[idx_v.at[i]],
                packed.at[:, pl.ds(i * _LW, _LW)],
                sem_g) for i in range(n_t)]
            for g in gs:
                g.wait()
            pltpu.async_copy(packed, out_hbm.at[pl.ds(base, _CHUNK)],
                             sem_o).wait()

    kern = pl.kernel(
        body,
        out_type=jax.ShapeDtypeStruct((B, n_t * _LW), jnp.float32),
        mesh=mesh,
        scratch_types=scratch,
    )
    return kern(XiW, *pairs)


def _tc_dense(G, Xv, bws, bbs, a1, tb1, w2, tb2, w3, tb3, w4, tb4, B, NF,
              EMB):
    BT = 512
    bf16 = jnp.bfloat16
    f32 = jnp.float32
    n1 = NF + 1
    GW = G.shape[1]

    def body(g_ref, xv_ref, bw1_ref, bw2_ref, bw3_ref,
             bb1_ref, bb2_ref, bb3_ref, a1_ref, tb1_ref, w2_ref, tb2_ref,
             w3_ref, tb3_ref, w4_ref, tb4_ref, out_ref):
        h = xv_ref[...].astype(bf16)
        h = jnp.maximum(jnp.dot(h, bw1_ref[...], preferred_element_type=f32)
                        + bb1_ref[...], 0.0)
        h = jnp.maximum(jnp.dot(h.astype(bf16), bw2_ref[...],
                                preferred_element_type=f32) + bb2_ref[...], 0.0)
        h = jnp.maximum(jnp.dot(h.astype(bf16), bw3_ref[...],
                                preferred_element_type=f32) + bb3_ref[...], 0.0)
        g = g_ref[...]
        embs = [g[:, i * _LW + (i % 2) * EMB: i * _LW + (i % 2) * EMB + EMB]
                for i in range(NF)]
        t = jnp.concatenate([h] + embs, axis=1).astype(bf16)
        t3 = t.reshape(BT, n1, EMB)
        z3 = lax.dot_general(t3, t3, (((2,), (2,)), ((0,), (0,))),
                             preferred_element_type=f32)
        zr = z3.reshape(BT, n1 * n1).astype(bf16)
        x1 = jnp.concatenate([h.astype(bf16), zr], axis=1)
        y = jnp.maximum(jnp.dot(x1, a1_ref[...], preferred_element_type=f32)
                        + tb1_ref[...], 0.0)
        y = jnp.maximum(jnp.dot(y.astype(bf16), w2_ref[...],
                                preferred_element_type=f32) + tb2_ref[...], 0.0)
        y = jnp.maximum(jnp.dot(y.astype(bf16), w3_ref[...],
                                preferred_element_type=f32) + tb3_ref[...], 0.0)
        y = jnp.dot(y.astype(bf16), w4_ref[...], preferred_element_type=f32) \
            + tb4_ref[...]
        out_ref[...] = jax.nn.sigmoid(y)

    grid = (B // BT,)
    full = lambda a: pl.BlockSpec(a.shape, lambda i: (0,) * a.ndim)
    in_specs = [
        pl.BlockSpec((BT, GW), lambda i: (i, 0)),
        pl.BlockSpec((BT, Xv.shape[1]), lambda i: (i, 0)),
        full(bws[0]), full(bws[1]), full(bws[2]),
        full(bbs[0]), full(bbs[1]), full(bbs[2]),
        full(a1), full(tb1), full(w2), full(tb2), full(w3), full(tb3),
        full(w4), full(tb4),
    ]
    out_spec = pl.BlockSpec((BT, 1), lambda i: (i, 0))
    return pl.pallas_call(
        body,
        grid=grid,
        in_specs=in_specs,
        out_specs=out_spec,
        out_shape=jax.ShapeDtypeStruct((B, 1), jnp.float32),
    )(G, Xv, *bws, *bbs, a1, tb1, w2, tb2, w3, tb3, w4, tb4)


def kernel(Xi, Xv, tables, projs, bot_w, bot_b, top_w, top_b):
    B, NF = Xi.shape
    V = tables[0].shape[0]
    EMB = projs[0].shape[0]
    n1 = NF + 1
    C = B // (_NW * _CHUNK)

    XiW = Xi.T.reshape(NF, _NW, C, _CHUNK).transpose(1, 2, 0, 3)
    tTs = [t.T for t in tables]
    pTs = [p.T for p in projs]

    bf16 = jnp.bfloat16
    bws = [w.T.astype(bf16) for w in bot_w]
    bbs = [b.reshape(1, -1) for b in bot_b]

    W1 = top_w[0]
    li, lj = np.tril_indices(n1, -1)
    rowidx = jnp.asarray(li * n1 + lj, dtype=jnp.int32)
    a_gram = jnp.zeros((n1 * n1, W1.shape[0]), dtype=jnp.float32)
    a_gram = a_gram.at[rowidx].set(W1[:, EMB:].T)
    a1 = jnp.concatenate([W1[:, :EMB].T, a_gram], axis=0).astype(bf16)
    tb1 = top_b[0].reshape(1, -1)
    w2 = top_w[1].T.astype(bf16)
    tb2 = top_b[1].reshape(1, -1)
    w3 = top_w[2].T.astype(bf16)
    tb3 = top_b[2].reshape(1, -1)
    w4 = top_w[3].T.astype(bf16)
    tb4 = top_b[3].reshape(1, -1)

    pairs = _tc_project(tTs, pTs, V, EMB)
    G = _sc_gather(XiW, pairs, B, C)
    return _tc_dense(G, Xv, bws, bbs, a1, tb1, w2, tb2, w3, tb3, w4, tb4,
                     B, NF, EMB)


# speedup vs baseline: 14.4688x; 1.0599x over previous
import functools

import numpy as np

import jax
import jax.numpy as jnp
from jax import lax
from jax.experimental import pallas as pl
from jax.experimental.pallas import tpu as pltpu
from jax.experimental.pallas import tpu_sc as plsc

_NW = 32
_CHUNK = 32
_LW = 128
_VB = 2048


def _tc_project(tTs, bds, V, EMB):
    n_t = len(tTs)
    n_p = n_t // 2
    bf16 = jnp.bfloat16
    f32 = jnp.float32

    def body(*refs):
        t_refs = refs[:n_t]
        b_refs = refs[n_t:n_t + n_p]
        o_refs = refs[n_t + n_p:]
        for p in range(n_p):
            lhs = jnp.concatenate(
                [t_refs[2 * p][...], t_refs[2 * p + 1][...]],
                axis=0).astype(bf16)
            o_refs[p][...] = lax.dot_general(
                lhs, b_refs[p][...], (((0,), (0,)), ((), ())),
                preferred_element_type=f32)

    grid = (pl.cdiv(V, _VB),)
    in_specs = (
        [pl.BlockSpec((t.shape[0], _VB), lambda i: (0, i)) for t in tTs]
        + [pl.BlockSpec(b.shape, lambda i: (0, 0)) for b in bds]
    )
    out_specs = [pl.BlockSpec((_VB, _LW), lambda i: (i, 0))
                 for _ in range(n_p)]
    return pl.pallas_call(
        body,
        grid=grid,
        in_specs=in_specs,
        out_specs=out_specs,
        out_shape=[jax.ShapeDtypeStruct((V, _LW), jnp.float32)
                   for _ in range(n_p)],
        compiler_params=pltpu.CompilerParams(
            vmem_limit_bytes=100 * 1024 * 1024),
    )(*tTs, *bds)


def _sc_gather(XiW, pairs, B, C):
    n_p = len(pairs)
    n_t = 2 * n_p
    mesh = plsc.VectorSubcoreMesh(core_axis_name="c", subcore_axis_name="s")
    scratch = [
        pltpu.VMEM((n_t, _CHUNK), jnp.int32),
        pltpu.VMEM((_CHUNK, n_t * _LW), jnp.float32),
        pltpu.SemaphoreType.DMA,
        pltpu.SemaphoreType.DMA,
        pltpu.SemaphoreType.DMA,
    ]

    def body(xiw_hbm, *refs):
        tabs = refs[:n_p]
        out_hbm = refs[n_p]
        idx_v = refs[n_p + 1]
        packed = refs[n_p + 2]
        sem_i, sem_g, sem_o = refs[n_p + 3:]
        w = lax.axis_index("s") * 2 + lax.axis_index("c")

        @pl.loop(0, C)
        def _(c):
            base = w * (C * _CHUNK) + c * _CHUNK
            pltpu.async_copy(xiw_hbm.at[w, c], idx_v, sem_i).wait()
            gs = [pltpu.async_copy(
                tabs[i // 2].at[idx_v.at[i]],
                packed.at[:, pl.ds(i * _LW, _LW)],
                sem_g) for i in range(n_t)]
            for g in gs:
                g.wait()
            pltpu.async_copy(packed, out_hbm.at[pl.ds(base, _CHUNK)],
                             sem_o).wait()

    kern = pl.kernel(
        body,
        out_type=jax.ShapeDtypeStruct((B, n_t * _LW), jnp.float32),
        mesh=mesh,
        scratch_types=scratch,
    )
    return kern(XiW, *pairs)


def _tc_dense(G, Xv, bws, bbs, a1, tb1, w2, tb2, w3, tb3, w4, tb4, B, NF,
              EMB):
    BT = 512
    bf16 = jnp.bfloat16
    f32 = jnp.float32
    n1 = NF + 1
    GW = G.shape[1]

    def body(g_ref, xv_ref, bw1_ref, bw2_ref, bw3_ref,
             bb1_ref, bb2_ref, bb3_ref, a1_ref, tb1_ref, w2_ref, tb2_ref,
             w3_ref, tb3_ref, w4_ref, tb4_ref, out_ref):
        h = xv_ref[...].astype(bf16)
        h = jnp.maximum(jnp.dot(h, bw1_ref[...], preferred_element_type=f32)
                        + bb1_ref[...], 0.0)
        h = jnp.maximum(jnp.dot(h.astype(bf16), bw2_ref[...],
                                preferred_element_type=f32) + bb2_ref[...], 0.0)
        h = jnp.maximum(jnp.dot(h.astype(bf16), bw3_ref[...],
                                preferred_element_type=f32) + bb3_ref[...], 0.0)
        g = g_ref[...]
        embs = [g[:, i * _LW + (i % 2) * EMB: i * _LW + (i % 2) * EMB + EMB]
                for i in range(NF)]
        t = jnp.concatenate([h] + embs, axis=1).astype(bf16)
        t3 = t.reshape(BT, n1, EMB)
        z3 = lax.dot_general(t3, t3, (((2,), (2,)), ((0,), (0,))),
                             preferred_element_type=f32)
        zr = z3.reshape(BT, n1 * n1).astype(bf16)
        x1 = jnp.concatenate([h.astype(bf16), zr], axis=1)
        y = jnp.maximum(jnp.dot(x1, a1_ref[...], preferred_element_type=f32)
                        + tb1_ref[...], 0.0)
        y = jnp.maximum(jnp.dot(y.astype(bf16), w2_ref[...],
                                preferred_element_type=f32) + tb2_ref[...], 0.0)
        y = jnp.maximum(jnp.dot(y.astype(bf16), w3_ref[...],
                                preferred_element_type=f32) + tb3_ref[...], 0.0)
        y = jnp.dot(y.astype(bf16), w4_ref[...], preferred_element_type=f32) \
            + tb4_ref[...]
        out_ref[...] = jax.nn.sigmoid(y)

    grid = (B // BT,)
    full = lambda a: pl.BlockSpec(a.shape, lambda i: (0,) * a.ndim)
    in_specs = [
        pl.BlockSpec((BT, GW), lambda i: (i, 0)),
        pl.BlockSpec((BT, Xv.shape[1]), lambda i: (i, 0)),
        full(bws[0]), full(bws[1]), full(bws[2]),
        full(bbs[0]), full(bbs[1]), full(bbs[2]),
        full(a1), full(tb1), full(w2), full(tb2), full(w3), full(tb3),
        full(w4), full(tb4),
    ]
    out_spec = pl.BlockSpec((BT, 1), lambda i: (i, 0))
    return pl.pallas_call(
        body,
        grid=grid,
        in_specs=in_specs,
        out_specs=out_spec,
        out_shape=jax.ShapeDtypeStruct((B, 1), jnp.float32),
    )(G, Xv, *bws, *bbs, a1, tb1, w2, tb2, w3, tb3, w4, tb4)


def kernel(Xi, Xv, tables, projs, bot_w, bot_b, top_w, top_b):
    B, NF = Xi.shape
    V = tables[0].shape[0]
    EMB = projs[0].shape[0]
    n1 = NF + 1
    C = B // (_NW * _CHUNK)

    XiW = Xi.T.reshape(NF, _NW, C, _CHUNK).transpose(1, 2, 0, 3)
    tTs = [t.T for t in tables]

    bf16 = jnp.bfloat16
    bds = []
    for p in range(NF // 2):
        ma = tables[2 * p].shape[1]
        mb = tables[2 * p + 1].shape[1]
        bd = jnp.zeros((ma + mb, 2 * EMB), dtype=bf16)
        bd = bd.at[:ma, :EMB].set(projs[2 * p].T.astype(bf16))
        bd = bd.at[ma:, EMB:].set(projs[2 * p + 1].T.astype(bf16))
        bds.append(bd)
    bws = [w.T.astype(bf16) for w in bot_w]
    bbs = [b.reshape(1, -1) for b in bot_b]

    W1 = top_w[0]
    li, lj = np.tril_indices(n1, -1)
    rowidx = jnp.asarray(li * n1 + lj, dtype=jnp.int32)
    a_gram = jnp.zeros((n1 * n1, W1.shape[0]), dtype=jnp.float32)
    a_gram = a_gram.at[rowidx].set(W1[:, EMB:].T)
    a1 = jnp.concatenate([W1[:, :EMB].T, a_gram], axis=0).astype(bf16)
    tb1 = top_b[0].reshape(1, -1)
    w2 = top_w[1].T.astype(bf16)
    tb2 = top_b[1].reshape(1, -1)
    w3 = top_w[2].T.astype(bf16)
    tb3 = top_b[2].reshape(1, -1)
    w4 = top_w[3].T.astype(bf16)
    tb4 = top_b[3].reshape(1, -1)

    pairs = _tc_project(tTs, bds, V, EMB)
    G = _sc_gather(XiW, pairs, B, C)
    return _tc_dense(G, Xv, bws, bbs, a1, tb1, w2, tb2, w3, tb3, w4, tb4,
                     B, NF, EMB)


# speedup vs baseline: 14.5641x; 1.0066x over previous
import functools

import numpy as np

import jax
import jax.numpy as jnp
from jax import lax
from jax.experimental import pallas as pl
from jax.experimental.pallas import tpu as pltpu
from jax.experimental.pallas import tpu_sc as plsc

_NW = 32
_CHUNK = 32
_LW = 128
_VB = 2048


def _tc_project(tTs, bds, V, EMB):
    n_t = len(tTs)
    n_p = n_t // 2
    bf16 = jnp.bfloat16
    f32 = jnp.float32

    def body(*refs):
        t_refs = refs[:n_t]
        b_refs = refs[n_t:n_t + n_p]
        o_refs = refs[n_t + n_p:]
        for p in range(n_p):
            lhs = jnp.concatenate(
                [t_refs[2 * p][...], t_refs[2 * p + 1][...]],
                axis=0).astype(bf16)
            o_refs[p][...] = lax.dot_general(
                lhs, b_refs[p][...], (((0,), (0,)), ((), ())),
                preferred_element_type=f32)

    grid = (pl.cdiv(V, _VB),)
    in_specs = (
        [pl.BlockSpec((t.shape[0], _VB), lambda i: (0, i)) for t in tTs]
        + [pl.BlockSpec(b.shape, lambda i: (0, 0)) for b in bds]
    )
    out_specs = [pl.BlockSpec((_VB, _LW), lambda i: (i, 0))
                 for _ in range(n_p)]
    return pl.pallas_call(
        body,
        grid=grid,
        in_specs=in_specs,
        out_specs=out_specs,
        out_shape=[jax.ShapeDtypeStruct((V, _LW), jnp.float32)
                   for _ in range(n_p)],
        compiler_params=pltpu.CompilerParams(
            vmem_limit_bytes=100 * 1024 * 1024),
    )(*tTs, *bds)


def _sc_gather(XiW, pairs, B, C):
    n_p = len(pairs)
    n_t = 2 * n_p
    mesh = plsc.VectorSubcoreMesh(core_axis_name="c", subcore_axis_name="s")
    scratch = [
        pltpu.VMEM((n_t, _CHUNK), jnp.int32),
        pltpu.VMEM((_CHUNK, n_t * _LW), jnp.float32),
        pltpu.SemaphoreType.DMA,
        pltpu.SemaphoreType.DMA,
        pltpu.SemaphoreType.DMA,
    ]

    nh = n_t // 2 * _LW

    def body(xiw_hbm, *refs):
        tabs = refs[:n_p]
        out_hbm = refs[n_p]
        idx_v = refs[n_p + 1]
        packed = refs[n_p + 2]
        sem_i, sem_g, sem_o = refs[n_p + 3:]
        w = lax.axis_index("s") * 2 + lax.axis_index("c")

        def _half(c, h):
            base = w * (C * _CHUNK) + c * _CHUNK
            lo = n_t // 2 * h
            gs = [pltpu.async_copy(
                tabs[i // 2].at[idx_v.at[i]],
                packed.at[:, pl.ds(i * _LW, _LW)],
                sem_g) for i in range(lo, lo + n_t // 2)]
            for g in gs:
                g.wait()
            return pltpu.async_copy(
                packed.at[:, pl.ds(nh * h, nh)],
                out_hbm.at[pl.ds(base, _CHUNK), pl.ds(nh * h, nh)],
                sem_o)

        @pl.loop(0, C)
        def _(c):
            pltpu.async_copy(xiw_hbm.at[w, c], idx_v, sem_i).wait()
            @pl.when(c > 0)
            def _():
                pltpu.make_async_copy(
                    packed.at[:, pl.ds(0, nh)],
                    out_hbm.at[pl.ds(0, _CHUNK), pl.ds(0, nh)],
                    sem_o).wait()
                pltpu.make_async_copy(
                    packed.at[:, pl.ds(0, nh)],
                    out_hbm.at[pl.ds(0, _CHUNK), pl.ds(0, nh)],
                    sem_o).wait()
            _half(c, 0)
            _half(c, 1)

        pltpu.make_async_copy(
            packed.at[:, pl.ds(0, nh)],
            out_hbm.at[pl.ds(0, _CHUNK), pl.ds(0, nh)], sem_o).wait()
        pltpu.make_async_copy(
            packed.at[:, pl.ds(0, nh)],
            out_hbm.at[pl.ds(0, _CHUNK), pl.ds(0, nh)], sem_o).wait()

    kern = pl.kernel(
        body,
        out_type=jax.ShapeDtypeStruct((B, n_t * _LW), jnp.float32),
        mesh=mesh,
        scratch_types=scratch,
    )
    return kern(XiW, *pairs)


def _tc_dense(G, Xv, bws, bbs, a1, tb1, w2, tb2, w3, tb3, w4, tb4, B, NF,
              EMB):
    BT = 512
    bf16 = jnp.bfloat16
    f32 = jnp.float32
    n1 = NF + 1
    GW = G.shape[1]

    def body(g_ref, xv_ref, bw1_ref, bw2_ref, bw3_ref,
             bb1_ref, bb2_ref, bb3_ref, a1_ref, tb1_ref, w2_ref, tb2_ref,
             w3_ref, tb3_ref, w4_ref, tb4_ref, out_ref):
        h = xv_ref[...].astype(bf16)
        h = jnp.maximum(jnp.dot(h, bw1_ref[...], preferred_element_type=f32)
                        + bb1_ref[...], 0.0)
        h = jnp.maximum(jnp.dot(h.astype(bf16), bw2_ref[...],
                                preferred_element_type=f32) + bb2_ref[...], 0.0)
        h = jnp.maximum(jnp.dot(h.astype(bf16), bw3_ref[...],
                                preferred_element_type=f32) + bb3_ref[...], 0.0)
        g = g_ref[...]
        embs = [g[:, i * _LW + (i % 2) * EMB: i * _LW + (i % 2) * EMB + EMB]
                for i in range(NF)]
        t = jnp.concatenate([h] + embs, axis=1).astype(bf16)
        t3 = t.reshape(BT, n1, EMB)
        z3 = lax.dot_general(t3, t3, (((2,), (2,)), ((0,), (0,))),
                             preferred_element_type=f32)
        zr = z3.reshape(BT, n1 * n1).astype(bf16)
        x1 = jnp.concatenate([h.astype(bf16), zr], axis=1)
        y = jnp.maximum(jnp.dot(x1, a1_ref[...], preferred_element_type=f32)
                        + tb1_ref[...], 0.0)
        y = jnp.maximum(jnp.dot(y.astype(bf16), w2_ref[...],
                                preferred_element_type=f32) + tb2_ref[...], 0.0)
        y = jnp.maximum(jnp.dot(y.astype(bf16), w3_ref[...],
                                preferred_element_type=f32) + tb3_ref[...], 0.0)
        y = jnp.dot(y.astype(bf16), w4_ref[...], preferred_element_type=f32) \
            + tb4_ref[...]
        out_ref[...] = jax.nn.sigmoid(y)

    grid = (B // BT,)
    full = lambda a: pl.BlockSpec(a.shape, lambda i: (0,) * a.ndim)
    in_specs = [
        pl.BlockSpec((BT, GW), lambda i: (i, 0)),
        pl.BlockSpec((BT, Xv.shape[1]), lambda i: (i, 0)),
        full(bws[0]), full(bws[1]), full(bws[2]),
        full(bbs[0]), full(bbs[1]), full(bbs[2]),
        full(a1), full(tb1), full(w2), full(tb2), full(w3), full(tb3),
        full(w4), full(tb4),
    ]
    out_spec = pl.BlockSpec((BT, 1), lambda i: (i, 0))
    return pl.pallas_call(
        body,
        grid=grid,
        in_specs=in_specs,
        out_specs=out_spec,
        out_shape=jax.ShapeDtypeStruct((B, 1), jnp.float32),
    )(G, Xv, *bws, *bbs, a1, tb1, w2, tb2, w3, tb3, w4, tb4)


def kernel(Xi, Xv, tables, projs, bot_w, bot_b, top_w, top_b):
    B, NF = Xi.shape
    V = tables[0].shape[0]
    EMB = projs[0].shape[0]
    n1 = NF + 1
    C = B // (_NW * _CHUNK)

    XiW = Xi.T.reshape(NF, _NW, C, _CHUNK).transpose(1, 2, 0, 3)
    tTs = [t.T for t in tables]

    bf16 = jnp.bfloat16
    bds = []
    for p in range(NF // 2):
        ma = tables[2 * p].shape[1]
        mb = tables[2 * p + 1].shape[1]
        bd = jnp.zeros((ma + mb, 2 * EMB), dtype=bf16)
        bd = bd.at[:ma, :EMB].set(projs[2 * p].T.astype(bf16))
        bd = bd.at[ma:, EMB:].set(projs[2 * p + 1].T.astype(bf16))
        bds.append(bd)
    bws = [w.T.astype(bf16) for w in bot_w]
    bbs = [b.reshape(1, -1) for b in bot_b]

    W1 = top_w[0]
    li, lj = np.tril_indices(n1, -1)
    rowidx = jnp.asarray(li * n1 + lj, dtype=jnp.int32)
    a_gram = jnp.zeros((n1 * n1, W1.shape[0]), dtype=jnp.float32)
    a_gram = a_gram.at[rowidx].set(W1[:, EMB:].T)
    a1 = jnp.concatenate([W1[:, :EMB].T, a_gram], axis=0).astype(bf16)
    tb1 = top_b[0].reshape(1, -1)
    w2 = top_w[1].T.astype(bf16)
    tb2 = top_b[1].reshape(1, -1)
    w3 = top_w[2].T.astype(bf16)
    tb3 = top_b[2].reshape(1, -1)
    w4 = top_w[3].T.astype(bf16)
    tb4 = top_b[3].reshape(1, -1)

    pairs = _tc_project(tTs, bds, V, EMB)
    G = _sc_gather(XiW, pairs, B, C)
    return _tc_dense(G, Xv, bws, bbs, a1, tb1, w2, tb2, w3, tb3, w4, tb4,
                     B, NF, EMB)
